# baseline (speedup 1.0000x reference)
import functools

import jax
import jax.numpy as jnp
import numpy as np
from jax.experimental import pallas as pl
from jax.experimental.pallas import tpu as pltpu

_RATE = 0.1


def _body(nn_ref, x_ref, o_ref, *, rows_per_blk, dd):
    s8 = rows_per_blk // 128
    row0 = pl.program_id(0) * rows_per_blk
    rows = (jax.lax.broadcasted_iota(jnp.int32, (s8, 128), 0) * 128
            + jax.lax.broadcasted_iota(jnp.int32, (s8, 128), 1) + row0)
    scale = jnp.full((s8, 128), dd[0], dtype=jnp.float32)
    e_k = nn_ref[0]
    for k in range(1, len(dd)):
        if dd[k] != 0.0:
            scale = scale + jnp.where(rows >= e_k, jnp.float32(dd[k]), 0.0)
        e_k = e_k + nn_ref[k]
    o_ref[...] = x_ref[...] * scale[:, :, None]


_drop_cache = {}


def _drop_vec(b):
    if b not in _drop_cache:
        keep = 1.0 - _RATE
        with jax.ensure_compile_time_eval():
            u = jax.random.uniform(jax.random.key(1), (b, 1), dtype=jnp.float32)
            drop = jnp.ones((b, 1), jnp.float32) / keep * jnp.floor(keep + u)
        _drop_cache[b] = np.asarray(drop)[:, 0]
    return _drop_cache[b]


def kernel(inputs, n_node):
    n, d = inputs.shape
    b = n_node.shape[0]
    drop = _drop_vec(b)
    dd = [float(drop[0])] + [float(drop[k] - drop[k - 1]) for k in range(1, b)]

    rows_per_blk = 4096
    grid = n // rows_per_blk
    x3 = inputs.reshape(n // 128, 128, d)
    out = pl.pallas_call(
        functools.partial(_body, rows_per_blk=rows_per_blk, dd=dd),
        grid=(grid,),
        in_specs=[
            pl.BlockSpec(memory_space=pltpu.SMEM),
            pl.BlockSpec((rows_per_blk // 128, 128, d), lambda i: (i, 0, 0)),
        ],
        out_specs=pl.BlockSpec((rows_per_blk // 128, 128, d), lambda i: (i, 0, 0)),
        out_shape=jax.ShapeDtypeStruct((n // 128, 128, d), inputs.dtype),
    )(n_node.astype(jnp.int32), x3)
    return out.reshape(n, d)


# speedup vs baseline: 1.0705x; 1.0705x over previous
import functools

import jax
import jax.numpy as jnp
import numpy as np
from jax.experimental import pallas as pl
from jax.experimental.pallas import tpu as pltpu

_RATE = 0.1


def _body(nn_ref, x_ref, o_ref, *, rows_per_blk, dd):
    s8 = rows_per_blk // 128
    row0 = pl.program_id(0) * rows_per_blk
    rows = (jax.lax.broadcasted_iota(jnp.int32, (s8, 128), 0) * 128
            + jax.lax.broadcasted_iota(jnp.int32, (s8, 128), 1) + row0)
    scale = jnp.full((s8, 128), dd[0], dtype=jnp.float32)
    e_k = nn_ref[0]
    for k in range(1, len(dd)):
        if dd[k] != 0.0:
            scale = scale + jnp.where(rows >= e_k, jnp.float32(dd[k]), 0.0)
        e_k = e_k + nn_ref[k]
    o_ref[...] = x_ref[...] * scale[:, :, None]


_drop_cache = {}


def _drop_vec(b):
    if b not in _drop_cache:
        keep = 1.0 - _RATE
        with jax.ensure_compile_time_eval():
            u = jax.random.uniform(jax.random.key(1), (b, 1), dtype=jnp.float32)
            drop = jnp.ones((b, 1), jnp.float32) / keep * jnp.floor(keep + u)
        _drop_cache[b] = np.asarray(drop)[:, 0]
    return _drop_cache[b]


def kernel(inputs, n_node):
    n, d = inputs.shape
    b = n_node.shape[0]
    drop = _drop_vec(b)
    dd = [float(drop[0])] + [float(drop[k] - drop[k - 1]) for k in range(1, b)]

    rows_per_blk = 8192
    grid = n // rows_per_blk
    x3 = inputs.reshape(n // 128, 128, d)
    out = pl.pallas_call(
        functools.partial(_body, rows_per_blk=rows_per_blk, dd=dd),
        grid=(grid,),
        in_specs=[
            pl.BlockSpec(memory_space=pltpu.SMEM),
            pl.BlockSpec((rows_per_blk // 128, 128, d), lambda i: (i, 0, 0)),
        ],
        out_specs=pl.BlockSpec((rows_per_blk // 128, 128, d), lambda i: (i, 0, 0)),
        out_shape=jax.ShapeDtypeStruct((n // 128, 128, d), inputs.dtype),
    )(n_node.astype(jnp.int32), x3)
    return out.reshape(n, d)
